# Initial kernel scaffold; baseline (speedup 1.0000x reference)
import functools

import jax
import jax.numpy as jnp
import numpy as np
from jax import lax
from jax.experimental import pallas as pl
from jax.experimental.pallas import tpu as pltpu
from jax.experimental.pallas import tpu_sc as plsc

_FIELD_DIMS = [38462] * 26
_OFFSETS = np.concatenate([[0], np.cumsum(_FIELD_DIMS[:-1])]).astype(np.int32)

NF = 26
D = 16
B = 16384
NC = 2
NS = 16
NW = NC * NS
ROWS_PER_W = B // NW
NB = 128
NCHUNK = ROWS_PER_W // NB
FLAT = NB * NF
G = FLAT // 128


def kernel(x, table, bias):
    x2d = x.reshape(B * NF // 128, 128)
    offs2d = jnp.asarray(np.tile(_OFFSETS, NB).reshape(G, 128))

    mesh = plsc.VectorSubcoreMesh(core_axis_name="c", subcore_axis_name="s")

    @functools.partial(
        pl.kernel,
        mesh=mesh,
        out_type=jax.ShapeDtypeStruct((B, D), jnp.float32),
        scratch_types=[
            pltpu.VMEM((G, 128), jnp.int32),
            pltpu.VMEM((G, 128), jnp.int32),
            pltpu.VMEM((FLAT, D), jnp.float32),
            pltpu.VMEM((NB, D), jnp.float32),
            pltpu.VMEM((D,), jnp.float32),
            pltpu.SemaphoreType.DMA,
        ],
    )
    def k(x_hbm, off_hbm, table_hbm, bias_hbm, out_hbm,
          idx_v, offs_v, rows_v, out_v, bias_v, sem):
        wid = lax.axis_index("s") * NC + lax.axis_index("c")
        pltpu.sync_copy(off_hbm, offs_v)
        pltpu.sync_copy(bias_hbm, bias_v)
        bias_vec = bias_v[...]

        def chunk_body(ci, carry):
            row0 = wid * ROWS_PER_W + ci * NB
            pltpu.sync_copy(
                x_hbm.at[pl.ds(wid * (ROWS_PER_W * NF // 128) + ci * G, G), :],
                idx_v)

            def add_off(g, c2):
                for l in range(128 // 16):
                    sl = pl.ds(l * 16, 16)
                    idx_v[g, sl] = idx_v[g, sl] + offs_v[g, sl]
                return c2
            lax.fori_loop(0, G, add_off, 0)

            copies = [
                pltpu.async_copy(table_hbm.at[idx_v.at[g]],
                                 rows_v.at[pl.ds(g * 128, 128)], sem)
                for g in range(G)
            ]
            for c in copies:
                c.wait()

            def row_body(b, c2):
                base = b * NF
                acc = rows_v[base, :]
                for f in range(1, NF):
                    acc = acc + rows_v[base + f, :]
                out_v[b, :] = acc + bias_vec
                return c2
            lax.fori_loop(0, NB, row_body, 0)

            pltpu.sync_copy(out_v, out_hbm.at[pl.ds(row0, NB), :])
            return carry

        lax.fori_loop(0, NCHUNK, chunk_body, 0)

    return k(x2d, offs2d, table, bias)


# speedup vs baseline: 1.1144x; 1.1144x over previous
import functools

import jax
import jax.numpy as jnp
import numpy as np
from jax import lax
from jax.experimental import pallas as pl
from jax.experimental.pallas import tpu as pltpu
from jax.experimental.pallas import tpu_sc as plsc

_FIELD_DIMS = [38462] * 26
_OFFSETS = np.concatenate([[0], np.cumsum(_FIELD_DIMS[:-1])]).astype(np.int32)

NF = 26
D = 16
B = 16384
NC = 2
NS = 16
NW = NC * NS
ROWS_PER_W = B // NW
NB = 128
NCHUNK = ROWS_PER_W // NB
FLAT = NB * NF
G = FLAT // 128


def kernel(x, table, bias):
    xf = x.reshape(B * NF)
    offsf = jnp.asarray(np.tile(_OFFSETS, NB))

    mesh = plsc.VectorSubcoreMesh(core_axis_name="c", subcore_axis_name="s")

    @functools.partial(
        pl.kernel,
        mesh=mesh,
        out_type=jax.ShapeDtypeStruct((B, D), jnp.float32),
        compiler_params=pltpu.CompilerParams(use_tc_tiling_on_sc=False),
        scratch_types=[
            pltpu.VMEM((FLAT,), jnp.int32),
            pltpu.VMEM((FLAT,), jnp.int32),
            pltpu.VMEM((FLAT, D), jnp.float32),
            pltpu.VMEM((NB, D), jnp.float32),
            pltpu.VMEM((D,), jnp.float32),
            pltpu.SemaphoreType.DMA,
        ],
    )
    def k(x_hbm, off_hbm, table_hbm, bias_hbm, out_hbm,
          idx_v, offs_v, rows_v, out_v, bias_v, sem):
        wid = lax.axis_index("s") * NC + lax.axis_index("c")
        pltpu.sync_copy(off_hbm, offs_v)
        pltpu.sync_copy(bias_hbm, bias_v)
        bias_vec = bias_v[...]

        def chunk_body(ci, carry):
            row0 = wid * ROWS_PER_W + ci * NB
            pltpu.sync_copy(
                x_hbm.at[pl.ds(wid * (ROWS_PER_W * NF) + ci * FLAT, FLAT)],
                idx_v)

            def add_off(t, c2):
                sl = pl.ds(t * 16, 16)
                idx_v[sl] = idx_v[sl] + offs_v[sl]
                return c2
            lax.fori_loop(0, FLAT // 16, add_off, 0)

            copies = [
                pltpu.async_copy(table_hbm.at[idx_v.at[pl.ds(g * 128, 128)]],
                                 rows_v.at[pl.ds(g * 128, 128)], sem)
                for g in range(G)
            ]
            for c in copies:
                c.wait()

            def row_body(b, c2):
                base = b * NF
                acc = rows_v[base, :]
                for f in range(1, NF):
                    acc = acc + rows_v[base + f, :]
                out_v[b, :] = acc + bias_vec
                return c2
            lax.fori_loop(0, NB, row_body, 0)

            pltpu.sync_copy(out_v, out_hbm.at[pl.ds(row0, NB), :])
            return carry

        lax.fori_loop(0, NCHUNK, chunk_body, 0)

    return k(xf, offsf, table, bias)


# speedup vs baseline: 1.1656x; 1.0460x over previous
import functools

import jax
import jax.numpy as jnp
import numpy as np
from jax import lax
from jax.experimental import pallas as pl
from jax.experimental.pallas import tpu as pltpu
from jax.experimental.pallas import tpu_sc as plsc

_FIELD_DIMS = [38462] * 26
_OFFSETS = np.concatenate([[0], np.cumsum(_FIELD_DIMS[:-1])]).astype(np.int32)

NF = 26
D = 16
B = 16384
NC = 2
NS = 16
NW = NC * NS
RW = B // NW
NQ = RW // 128
L = 16


def kernel(x, table, bias):
    xT = x.T

    mesh = plsc.VectorSubcoreMesh(core_axis_name="c", subcore_axis_name="s")

    @functools.partial(
        pl.kernel,
        mesh=mesh,
        out_type=jax.ShapeDtypeStruct((B, D), jnp.float32),
        compiler_params=pltpu.CompilerParams(use_tc_tiling_on_sc=False),
        scratch_types=[
            pltpu.VMEM((NF * NQ, 128), jnp.int32),
            pltpu.VMEM((RW, D), jnp.float32),
            pltpu.VMEM((D,), jnp.float32),
            pltpu.SemaphoreType.DMA,
            pltpu.SemaphoreType.DMA,
        ],
    )
    def k(xT_hbm, table_hbm, bias_hbm, out_hbm, idxT, acc, bias_v, sem, sem2):
        wid = lax.axis_index("s") * NC + lax.axis_index("c")
        base = wid * RW
        pltpu.sync_copy(bias_hbm, bias_v)

        stages = [
            pltpu.async_copy(
                xT_hbm.at[f, pl.ds(base + q * 128, 128)],
                idxT.at[f * NQ + q], sem2)
            for f in range(NF) for q in range(NQ)
        ]

        bias_vec = bias_v[...]
        def seed(b, c2):
            acc[b, :] = bias_vec
            return c2
        lax.fori_loop(0, RW, seed, 0)

        for c in stages:
            c.wait()

        for f in range(NF):
            off = int(_OFFSETS[f])
            def add_off(t, c2, f=f, off=off):
                q = t // 8
                col = (t % 8) * L
                r = f * NQ + q
                idxT[r, pl.ds(col, L)] = idxT[r, pl.ds(col, L)] + off
                return c2
            lax.fori_loop(0, 8 * NQ, add_off, 0)

        copies = [
            pltpu.async_copy(table_hbm.at[idxT.at[f * NQ + q]],
                             acc.at[pl.ds(q * 128, 128)], sem, add=True)
            for f in range(NF) for q in range(NQ)
        ]
        for c in copies:
            c.wait()

        pltpu.sync_copy(acc, out_hbm.at[pl.ds(base, RW)])

    return k(xT, table, bias)
